# baseline (speedup 1.0000x reference)
import functools

import jax
import jax.numpy as jnp
from jax import lax
from jax.experimental import pallas as pl
from jax.experimental.pallas import tpu as pltpu
from jax.experimental.pallas import tpu_sc as plsc

NC = 2
NS = 16
NW = NC * NS
LANES = 16
PDIM = 7


@functools.cache
def _make_sc_kernel(B, L, D):
    LW = L // NW
    DJ = D // LANES
    mesh = plsc.VectorSubcoreMesh(core_axis_name="c", subcore_axis_name="s")

    @functools.partial(
        pl.kernel,
        mesh=mesh,
        out_type=jax.ShapeDtypeStruct((B, L, D), jnp.float32),
        compiler_params=pltpu.CompilerParams(use_tc_tiling_on_sc=False),
        scratch_types=[
            pltpu.VMEM((B, LW), jnp.int32),
            pltpu.VMEM((B, LW, LANES), jnp.float32),
            pltpu.VMEM((LW, D), jnp.float32),
            pltpu.VMEM((PDIM, D), jnp.float32),
            pltpu.VMEM((D,), jnp.float32),
            pltpu.VMEM((LW, D), jnp.float32),
            pltpu.SemaphoreType.DMA,
        ],
    )
    def k(ids_hbm, pros_hbm, table_hbm, pos_hbm, projw_hbm, projb_hbm,
          out_hbm, ids_v, pros_v, posb_v, w_v, b_v, rows_v, sem):
        wid = lax.axis_index("s") * NC + lax.axis_index("c")
        l0 = wid * LW

        pltpu.sync_copy(ids_hbm.at[wid], ids_v)
        pltpu.sync_copy(pros_hbm.at[wid], pros_v)
        pltpu.sync_copy(pos_hbm.at[pl.ds(l0, LW)], posb_v)
        pltpu.sync_copy(projw_hbm, w_v)
        pltpu.sync_copy(projb_hbm, b_v)

        def fold_i(i, _):
            def fold_j(j, _):
                sl = pl.ds(j * LANES, LANES)
                posb_v[i, sl] = posb_v[i, sl] + b_v[sl]
                return _
            return lax.fori_loop(0, DJ, fold_j, _)
        lax.fori_loop(0, LW, fold_i, None)

        def body_b(b, _):
            pltpu.async_copy(table_hbm.at[ids_v.at[b]], rows_v, sem).wait()

            def body_i(i, _):
                pv = pros_v[b, i, :]
                p0 = pv[0]
                p1 = pv[1]
                p2 = pv[2]
                p3 = pv[3]
                p4 = pv[4]
                p5 = pv[5]
                p6 = pv[6]

                def body_j(j, _):
                    sl = pl.ds(j * LANES, LANES)
                    acc = rows_v[i, sl] + posb_v[i, sl]
                    acc = acc + p0 * w_v[0, sl]
                    acc = acc + p1 * w_v[1, sl]
                    acc = acc + p2 * w_v[2, sl]
                    acc = acc + p3 * w_v[3, sl]
                    acc = acc + p4 * w_v[4, sl]
                    acc = acc + p5 * w_v[5, sl]
                    acc = acc + p6 * w_v[6, sl]
                    rows_v[i, sl] = acc
                    return _
                return lax.fori_loop(0, DJ, body_j, _)
            lax.fori_loop(0, LW, body_i, None)

            pltpu.sync_copy(rows_v, out_hbm.at[b, pl.ds(l0, LW)])
            return _
        lax.fori_loop(0, B, body_b, None)

    return k


def kernel(token_ids, prosody_features, token_table, pos_table, proj_w,
           proj_b):
    B, L = token_ids.shape
    D = token_table.shape[1]
    LW = L // NW
    ids_prep = (token_ids.astype(jnp.int32)
                .reshape(B, NW, LW).transpose(1, 0, 2))
    pros_prep = jnp.pad(prosody_features, ((0, 0), (0, 0), (0, LANES - PDIM)))
    pros_prep = pros_prep.reshape(B, NW, LW, LANES).transpose(1, 0, 2, 3)
    k = _make_sc_kernel(B, L, D)
    return k(ids_prep, pros_prep, token_table, pos_table, proj_w, proj_b)
